# baseline (speedup 1.0000x reference)
import jax
import jax.numpy as jnp
from jax import lax
from jax.experimental import pallas as pl
from jax.experimental.pallas import tpu as pltpu
from jax.experimental.pallas import tpu_sc as plsc

MAX_SEG = 196
EMB = 768
F32 = jnp.float32


def _conv1_body(pat_ref, w_ref, b_ref, y_ref, st_ref):
    i = pl.program_id(0)
    y = jnp.dot(pat_ref[...], w_ref[...], preferred_element_type=F32)
    y = y + b_ref[...]
    s1 = jnp.sum(y, axis=0, keepdims=True)
    s2 = jnp.sum(y * y, axis=0, keepdims=True)
    st = jnp.concatenate([s1, s2, jnp.zeros((6, 64), F32)], axis=0)

    @pl.when(i == 0)
    def _():
        st_ref[...] = jnp.zeros_like(st_ref)

    st_ref[...] += st
    y_ref[...] = y


def _conv1_call(pat, w1, b1):
    n = pat.shape[0]
    blk = 2048
    grid = n // blk
    return pl.pallas_call(
        _conv1_body,
        grid=(grid,),
        in_specs=[
            pl.BlockSpec((blk, 147), lambda i: (i, 0)),
            pl.BlockSpec((147, 64), lambda i: (0, 0)),
            pl.BlockSpec((1, 64), lambda i: (0, 0)),
        ],
        out_specs=[
            pl.BlockSpec((blk, 64), lambda i: (i, 0)),
            pl.BlockSpec((8, 64), lambda i: (0, 0)),
        ],
        out_shape=[
            jax.ShapeDtypeStruct((n, 64), F32),
            jax.ShapeDtypeStruct((8, 64), F32),
        ],
    )(pat, w1, b1)


def _conv2_body(q_ref, w_ref, ab_ref, b2_ref, y_ref, st_ref, col_ref):
    b = pl.program_id(0)
    cix = pl.program_id(1)
    h0 = cix * 8
    a1 = ab_ref[0:1, :]
    c1 = ab_ref[1:2, :]
    for dy in range(3):
        p, a = dy % 2, dy // 2
        blk = q_ref[0, p, pl.ds(h0 + a, 8), pl.ds(0, 56), :]
        blk = jnp.maximum(blk * a1[0] + c1[0], 0.0).reshape(448, 128)
        col_ref[:, 192 * dy:192 * dy + 128] = blk
        blk2 = q_ref[0, p, pl.ds(h0 + a, 8), pl.ds(1, 56), 0:64]
        blk2 = jnp.maximum(blk2 * a1[0, 0:64] + c1[0, 0:64], 0.0).reshape(448, 64)
        col_ref[:, 192 * dy + 128:192 * dy + 192] = blk2
    y = jnp.dot(col_ref[...], w_ref[...], preferred_element_type=F32)
    y = y + b2_ref[...]
    s1 = jnp.sum(y, axis=0, keepdims=True)
    s2 = jnp.sum(y * y, axis=0, keepdims=True)
    st = jnp.concatenate([s1, s2, jnp.zeros((6, EMB), F32)], axis=0)

    @pl.when(jnp.logical_and(b == 0, cix == 0))
    def _():
        st_ref[...] = jnp.zeros_like(st_ref)

    st_ref[...] += st
    y_ref[...] = y.reshape(1, 8, 56, EMB)


def _conv2_call(q2, w2, ab, b2):
    return pl.pallas_call(
        _conv2_body,
        grid=(8, 7),
        in_specs=[
            pl.BlockSpec((1, 2, 57, 57, 128), lambda b, c: (b, 0, 0, 0, 0)),
            pl.BlockSpec((576, EMB), lambda b, c: (0, 0)),
            pl.BlockSpec((2, 128), lambda b, c: (0, 0)),
            pl.BlockSpec((1, EMB), lambda b, c: (0, 0)),
        ],
        out_specs=[
            pl.BlockSpec((1, 8, 56, EMB), lambda b, c: (b, c, 0, 0)),
            pl.BlockSpec((8, EMB), lambda b, c: (0, 0)),
        ],
        out_shape=[
            jax.ShapeDtypeStruct((8, 56, 56, EMB), F32),
            jax.ShapeDtypeStruct((8, EMB), F32),
        ],
        scratch_shapes=[pltpu.VMEM((448, 576), F32)],
    )(q2, w2, ab, b2)


def _seg_body(x_ref, seg_ref, prm_ref, out_ref, segb, buf, prmv, acc):
    c = lax.axis_index("c")
    s = lax.axis_index("s")
    img_l = s // 4
    part = s % 4
    img = 4 * c + img_l

    pltpu.sync_copy(prm_ref, prmv)

    base = part * 784
    for h in range(2):
        col0 = 384 * h

        def _zrow(r, carry):
            for g in range(24):
                acc[r, pl.ds(16 * g, 16)] = jnp.zeros((16,), F32)
            return carry

        lax.fori_loop(0, 200, _zrow, 0)
        iota = lax.broadcasted_iota(jnp.int32, (16,), 0)

        def _chunk(k, carry):
            pltpu.sync_copy(seg_ref.at[img, part * 14 + k], segb)
            pltpu.sync_copy(
                x_ref.at[img, pl.ds(base + 56 * k, 56), pl.ds(col0, 384)],
                buf)

            def _row(r, c2):
                sv = segb[r, :]
                for g in range(24):
                    v = buf[r, pl.ds(16 * g, 16)]
                    sg = prmv[0, pl.ds(col0 + 16 * g, 16)]
                    tg = prmv[1, pl.ds(col0 + 16 * g, 16)]
                    v = jnp.maximum(v * sg + tg, 0.0)
                    plsc.addupdate_scatter(acc, [sv, iota + 16 * g], v)
                return c2

            lax.fori_loop(0, 56, _row, 0)
            return carry

        lax.fori_loop(0, 14, _chunk, 0)
        pltpu.sync_copy(acc, out_ref.at[h, img, part])


def _seg_call(x2f, seg28, prm):
    mesh = plsc.VectorSubcoreMesh(core_axis_name="c", subcore_axis_name="s",
                                  num_cores=2, num_subcores=16)
    fn = pl.kernel(
        _seg_body,
        out_type=jax.ShapeDtypeStruct((2, 8, 4, 200, 384), F32),
        mesh=mesh,
        compiler_params=pltpu.CompilerParams(needs_layout_passes=False),
        scratch_types=[
            pltpu.VMEM((56, 16), jnp.int32),
            pltpu.VMEM((56, 384), F32),
            pltpu.VMEM((2, EMB), F32),
            pltpu.VMEM((200, 384), F32),
        ],
    )
    return fn(x2f, seg28, prm)


def _fin_body(p_ref, seg_ref, aa_ref, cn_ref, pw_ref, pb_ref, out_ref):
    def cnt_step(k, acc):
        sl = seg_ref[0, k]
        iota = lax.broadcasted_iota(jnp.int32, (256, 8), 0)
        return acc + (iota == sl).astype(F32)

    acc = lax.fori_loop(0, 392, cnt_step, jnp.zeros((256, 8), F32))
    cnt = jnp.maximum(jnp.sum(acc, axis=1, keepdims=True), 1.0)
    sums = (p_ref[0, 0, 0] + p_ref[0, 0, 1]
            + p_ref[0, 0, 2] + p_ref[0, 0, 3])[0:MAX_SEG]
    pos = (cn_ref[0][:, 0:1] * pw_ref[0:1, :]
           + cn_ref[0][:, 1:2] * pw_ref[1:2, :] + pb_ref[...])
    out_ref[0] = sums * aa_ref[...] / cnt[0:MAX_SEG, :] + pos


def _fin_call(parts, seg4, absa2, cn3, pw, pb):
    return pl.pallas_call(
        _fin_body,
        grid=(8, 2),
        in_specs=[
            pl.BlockSpec((1, 1, 4, 200, 384), lambda b, h: (h, b, 0, 0, 0)),
            pl.BlockSpec((1, 392, 1, 8), lambda b, h: (b, 0, 0, 0)),
            pl.BlockSpec((1, 384), lambda b, h: (0, h)),
            pl.BlockSpec((1, MAX_SEG, 2), lambda b, h: (b, 0, 0)),
            pl.BlockSpec((2, 384), lambda b, h: (0, h)),
            pl.BlockSpec((1, 384), lambda b, h: (0, h)),
        ],
        out_specs=pl.BlockSpec((1, MAX_SEG, 384), lambda b, h: (b, 0, h)),
        out_shape=jax.ShapeDtypeStruct((8, MAX_SEG, EMB), F32),
    )(parts, seg4, absa2, cn3, pw, pb)


def kernel(img, segments, centroid_coords, conv1_w, conv1_b, bn1_g, bn1_b,
           conv2_w, conv2_b, bn2_g, bn2_b, pos_w, pos_b):
    B, _, H, W = img.shape
    eps = 1e-5

    imgh = jnp.transpose(img.astype(F32), (0, 2, 3, 1))
    pat = lax.conv_general_dilated_patches(
        imgh, (7, 7), (2, 2), [(3, 3), (3, 3)],
        dimension_numbers=("NHWC", "HWIO", "NHWC"))
    pat = pat.reshape(B * 112 * 112, 147)
    w1 = jnp.transpose(conv1_w, (1, 2, 3, 0)).reshape(147, 64)

    x1, st1 = _conv1_call(pat, w1, conv1_b.reshape(1, 64))

    n1 = float(B * 112 * 112)
    mean1 = st1[0] / n1
    var1 = st1[1] / n1 - mean1 * mean1
    a1 = bn1_g / jnp.sqrt(var1 + eps)
    c1 = bn1_b - mean1 * a1
    ab = jnp.tile(jnp.stack([a1, c1]), (1, 2))

    x1 = x1.reshape(B, 112, 112, 64)
    x1p = jnp.pad(x1, ((0, 0), (1, 1), (1, 1), (0, 0)))
    q2 = x1p.reshape(B, 57, 2, 57, 2, 64).transpose(0, 2, 1, 3, 4, 5)
    q2 = q2.reshape(B, 2, 57, 57, 128)

    w2 = jnp.transpose(conv2_w, (2, 3, 1, 0)).reshape(576, EMB)
    x2, st2 = _conv2_call(q2, w2, ab, conv2_b.reshape(1, EMB))

    n2 = float(B * 56 * 56)
    mean2 = st2[0] / n2
    var2 = st2[1] / n2 - mean2 * mean2
    ra2 = bn2_g / jnp.sqrt(var2 + eps)
    c2 = bn2_b - mean2 * ra2
    absa2 = jnp.maximum(jnp.abs(ra2), 1e-30)
    sg2 = jnp.sign(ra2)
    t2 = c2 / absa2
    prm = jnp.stack([sg2, t2])

    seg_ds = segments[:, ::4, ::4].astype(jnp.int32)
    segx = jnp.broadcast_to(seg_ds.reshape(B, 56, 56, 1),
                            (B, 56, 56, 16)).reshape(B, 56, 56, 16)
    x2f = x2.reshape(B, 56 * 56, EMB)

    parts = _seg_call(x2f, segx, prm)

    cn = (centroid_coords.astype(F32) / jnp.array([float(W), float(H)], F32))
    out = _fin_call(parts,
                    seg_ds.reshape(B, 392, 1, 8),
                    absa2.reshape(1, EMB),
                    cn.reshape(B, MAX_SEG, 2),
                    pos_w, pos_b.reshape(1, EMB))
    return out


# speedup vs baseline: 1.0225x; 1.0225x over previous
import jax
import jax.numpy as jnp
from jax import lax
from jax.experimental import pallas as pl
from jax.experimental.pallas import tpu as pltpu
from jax.experimental.pallas import tpu_sc as plsc

MAX_SEG = 196
EMB = 768
F32 = jnp.float32


def _conv1_body(pat_ref, w_ref, b_ref, y_ref, st_ref):
    i = pl.program_id(0)
    y = jnp.dot(pat_ref[...], w_ref[...], preferred_element_type=F32)
    y = y + b_ref[...]
    s1 = jnp.sum(y, axis=0, keepdims=True)
    s2 = jnp.sum(y * y, axis=0, keepdims=True)
    st = jnp.concatenate([s1, s2, jnp.zeros((6, 64), F32)], axis=0)

    @pl.when(i == 0)
    def _():
        st_ref[...] = jnp.zeros_like(st_ref)

    st_ref[...] += st
    y_ref[...] = y


def _conv1_call(pat, w1, b1):
    n = pat.shape[0]
    blk = 2048
    grid = n // blk
    return pl.pallas_call(
        _conv1_body,
        grid=(grid,),
        in_specs=[
            pl.BlockSpec((blk, 147), lambda i: (i, 0)),
            pl.BlockSpec((147, 64), lambda i: (0, 0)),
            pl.BlockSpec((1, 64), lambda i: (0, 0)),
        ],
        out_specs=[
            pl.BlockSpec((blk, 64), lambda i: (i, 0)),
            pl.BlockSpec((8, 64), lambda i: (0, 0)),
        ],
        out_shape=[
            jax.ShapeDtypeStruct((n, 64), F32),
            jax.ShapeDtypeStruct((8, 64), F32),
        ],
    )(pat, w1, b1)


def _conv2_body(q_ref, w_ref, ab_ref, b2_ref, y_ref, st_ref, col_ref):
    b = pl.program_id(0)
    cix = pl.program_id(1)
    h0 = cix * 8
    a1 = ab_ref[0:1, :]
    c1 = ab_ref[1:2, :]
    for dy in range(3):
        p, a = dy % 2, dy // 2
        blk = q_ref[0, p, pl.ds(h0 + a, 8), pl.ds(0, 56), :]
        blk = jnp.maximum(blk * a1[0] + c1[0], 0.0).reshape(448, 128)
        col_ref[:, 192 * dy:192 * dy + 128] = blk
        blk2 = q_ref[0, p, pl.ds(h0 + a, 8), pl.ds(1, 56), 0:64]
        blk2 = jnp.maximum(blk2 * a1[0, 0:64] + c1[0, 0:64], 0.0).reshape(448, 64)
        col_ref[:, 192 * dy + 128:192 * dy + 192] = blk2
    y = jnp.dot(col_ref[...], w_ref[...], preferred_element_type=F32)
    y = y + b2_ref[...]
    s1 = jnp.sum(y, axis=0, keepdims=True)
    s2 = jnp.sum(y * y, axis=0, keepdims=True)
    st = jnp.concatenate([s1, s2, jnp.zeros((6, EMB), F32)], axis=0)

    @pl.when(jnp.logical_and(b == 0, cix == 0))
    def _():
        st_ref[...] = jnp.zeros_like(st_ref)

    st_ref[...] += st
    y_ref[...] = y.reshape(1, 8, 56, EMB)


def _conv2_call(q2, w2, ab, b2):
    return pl.pallas_call(
        _conv2_body,
        grid=(8, 7),
        in_specs=[
            pl.BlockSpec((1, 2, 57, 57, 128), lambda b, c: (b, 0, 0, 0, 0)),
            pl.BlockSpec((576, EMB), lambda b, c: (0, 0)),
            pl.BlockSpec((2, 128), lambda b, c: (0, 0)),
            pl.BlockSpec((1, EMB), lambda b, c: (0, 0)),
        ],
        out_specs=[
            pl.BlockSpec((1, 8, 56, EMB), lambda b, c: (b, c, 0, 0)),
            pl.BlockSpec((8, EMB), lambda b, c: (0, 0)),
        ],
        out_shape=[
            jax.ShapeDtypeStruct((8, 56, 56, EMB), F32),
            jax.ShapeDtypeStruct((8, EMB), F32),
        ],
        scratch_shapes=[pltpu.VMEM((448, 576), F32)],
    )(q2, w2, ab, b2)


def _seg_body(x_ref, seg_ref, prm_ref, out_ref, segb, buf, prmv, acc):
    c = lax.axis_index("c")
    s = lax.axis_index("s")
    img_l = s // 4
    part = s % 4
    img = 4 * c + img_l

    pltpu.sync_copy(prm_ref, prmv)
    pltpu.sync_copy(seg_ref.at[img, part], segb)

    base = part * 784
    for h in range(2):
        col0 = 384 * h

        def _zrow(r, carry):
            for g in range(24):
                acc[r, pl.ds(16 * g, 16)] = jnp.zeros((16,), F32)
            return carry

        lax.fori_loop(0, 200, _zrow, 0)
        iota = lax.broadcasted_iota(jnp.int32, (16,), 0)

        def _chunk(k, carry):
            pltpu.sync_copy(
                x_ref.at[img, pl.ds(base + 56 * k, 56), pl.ds(col0, 384)],
                buf)

            def _row(r, c2):
                idx = 56 * k + r
                sv = segb[idx // 8, pl.ds(16 * (idx % 8), 16)]
                for g in range(24):
                    v = buf[r, pl.ds(16 * g, 16)]
                    sg = prmv[0, pl.ds(col0 + 16 * g, 16)]
                    tg = prmv[1, pl.ds(col0 + 16 * g, 16)]
                    v = jnp.maximum(v * sg + tg, 0.0)
                    plsc.addupdate_scatter(acc, [sv, iota + 16 * g], v)
                return c2

            lax.fori_loop(0, 56, _row, 0)
            return carry

        lax.fori_loop(0, 14, _chunk, 0)
        pltpu.sync_copy(acc, out_ref.at[h, img, part])


def _seg_call(x2f, seg28, prm):
    mesh = plsc.VectorSubcoreMesh(core_axis_name="c", subcore_axis_name="s",
                                  num_cores=2, num_subcores=16)
    fn = pl.kernel(
        _seg_body,
        out_type=jax.ShapeDtypeStruct((2, 8, 4, 200, 384), F32),
        mesh=mesh,
        compiler_params=pltpu.CompilerParams(needs_layout_passes=False),
        scratch_types=[
            pltpu.VMEM((98, 128), jnp.int32),
            pltpu.VMEM((56, 384), F32),
            pltpu.VMEM((2, EMB), F32),
            pltpu.VMEM((200, 384), F32),
        ],
    )
    return fn(x2f, seg28, prm)


def _fin_body(p_ref, seg_ref, aa_ref, cn_ref, pw_ref, pb_ref, out_ref):
    def cnt_step(k, acc):
        sl = seg_ref[0, k]
        iota = lax.broadcasted_iota(jnp.int32, (256, 8), 0)
        return acc + (iota == sl).astype(F32)

    acc = lax.fori_loop(0, 392, cnt_step, jnp.zeros((256, 8), F32))
    cnt = jnp.maximum(jnp.sum(acc, axis=1, keepdims=True), 1.0)
    sums = (p_ref[0, 0, 0] + p_ref[0, 0, 1]
            + p_ref[0, 0, 2] + p_ref[0, 0, 3])[0:MAX_SEG]
    pos = (cn_ref[0][:, 0:1] * pw_ref[0:1, :]
           + cn_ref[0][:, 1:2] * pw_ref[1:2, :] + pb_ref[...])
    out_ref[0] = sums * aa_ref[...] / cnt[0:MAX_SEG, :] + pos


def _fin_call(parts, seg4, absa2, cn3, pw, pb):
    return pl.pallas_call(
        _fin_body,
        grid=(8, 2),
        in_specs=[
            pl.BlockSpec((1, 1, 4, 200, 384), lambda b, h: (h, b, 0, 0, 0)),
            pl.BlockSpec((1, 392, 1, 8), lambda b, h: (b, 0, 0, 0)),
            pl.BlockSpec((1, 384), lambda b, h: (0, h)),
            pl.BlockSpec((1, MAX_SEG, 2), lambda b, h: (b, 0, 0)),
            pl.BlockSpec((2, 384), lambda b, h: (0, h)),
            pl.BlockSpec((1, 384), lambda b, h: (0, h)),
        ],
        out_specs=pl.BlockSpec((1, MAX_SEG, 384), lambda b, h: (b, 0, h)),
        out_shape=jax.ShapeDtypeStruct((8, MAX_SEG, EMB), F32),
    )(parts, seg4, absa2, cn3, pw, pb)


def kernel(img, segments, centroid_coords, conv1_w, conv1_b, bn1_g, bn1_b,
           conv2_w, conv2_b, bn2_g, bn2_b, pos_w, pos_b):
    B, _, H, W = img.shape
    eps = 1e-5

    imgh = jnp.transpose(img.astype(F32), (0, 2, 3, 1))
    pat = lax.conv_general_dilated_patches(
        imgh, (7, 7), (2, 2), [(3, 3), (3, 3)],
        dimension_numbers=("NHWC", "HWIO", "NHWC"))
    pat = pat.reshape(B * 112 * 112, 147)
    w1 = jnp.transpose(conv1_w, (1, 2, 3, 0)).reshape(147, 64)

    x1, st1 = _conv1_call(pat, w1, conv1_b.reshape(1, 64))

    n1 = float(B * 112 * 112)
    mean1 = st1[0] / n1
    var1 = st1[1] / n1 - mean1 * mean1
    a1 = bn1_g / jnp.sqrt(var1 + eps)
    c1 = bn1_b - mean1 * a1
    ab = jnp.tile(jnp.stack([a1, c1]), (1, 2))

    x1 = x1.reshape(B, 112, 112, 64)
    x1p = jnp.pad(x1, ((0, 0), (1, 1), (1, 1), (0, 0)))
    q2 = x1p.reshape(B, 57, 2, 57, 2, 64).transpose(0, 2, 1, 3, 4, 5)
    q2 = q2.reshape(B, 2, 57, 57, 128)

    w2 = jnp.transpose(conv2_w, (2, 3, 1, 0)).reshape(576, EMB)
    x2, st2 = _conv2_call(q2, w2, ab, conv2_b.reshape(1, EMB))

    n2 = float(B * 56 * 56)
    mean2 = st2[0] / n2
    var2 = st2[1] / n2 - mean2 * mean2
    ra2 = bn2_g / jnp.sqrt(var2 + eps)
    c2 = bn2_b - mean2 * ra2
    absa2 = jnp.maximum(jnp.abs(ra2), 1e-30)
    sg2 = jnp.sign(ra2)
    t2 = c2 / absa2
    prm = jnp.stack([sg2, t2])

    seg_ds = segments[:, ::4, ::4].astype(jnp.int32)
    segx = jnp.broadcast_to(seg_ds.reshape(B, 4, 784, 1),
                            (B, 4, 784, 16)).reshape(B, 4, 98, 128)
    x2f = x2.reshape(B, 56 * 56, EMB)

    parts = _seg_call(x2f, segx, prm)

    cn = (centroid_coords.astype(F32) / jnp.array([float(W), float(H)], F32))
    out = _fin_call(parts,
                    seg_ds.reshape(B, 392, 1, 8),
                    absa2.reshape(1, EMB),
                    cn.reshape(B, MAX_SEG, 2),
                    pos_w, pos_b.reshape(1, EMB))
    return out
